# baseline (speedup 1.0000x reference)
import functools

import jax
import jax.numpy as jnp
from jax import lax
from jax.experimental import pallas as pl
from jax.experimental.pallas import tpu as pltpu
from jax.experimental.pallas import tpu_sc as plsc

B = 16384
NU = 1000000
NI = 100000
NL = 100
NCAT = 1000
D = 64
H = 32

NC = 2
NS = 16
NW = NC * NS
BPW = B // NW

TILE = 512
CKH = 4096


def _proj_body(xt_ref, w_ref, out_ref):
  h = lax.dot_general(
      xt_ref[...], w_ref[...], (((0,), (0,)), ((), ())),
      preferred_element_type=jnp.float32)
  u = lax.bitcast_convert_type(h, jnp.uint32)
  u = u + jnp.uint32(0x7FFF) + ((u >> 16) & jnp.uint32(1))
  b = u >> 16
  b2 = b.reshape(CKH, 2, 2 * D)
  out_ref[...] = b2[:, 0, :] | (b2[:, 1, :] << 16)


def _make_proj(n_pairs):
  grid = (n_pairs + CKH - 1) // CKH
  return pl.pallas_call(
      _proj_body,
      grid=(grid,),
      in_specs=[
          pl.BlockSpec((D, 2 * CKH), lambda i: (0, i)),
          pl.BlockSpec((D, 2 * D), lambda i: (0, 0)),
      ],
      out_specs=pl.BlockSpec((CKH, 2 * D), lambda i: (i, 0)),
      out_shape=jax.ShapeDtypeStruct((n_pairs, 2 * D), jnp.uint32),
      compiler_params=pltpu.CompilerParams(
          dimension_semantics=("arbitrary",)),
  )


_proj_u = _make_proj(NU // 2)
_proj_i = _make_proj(NI // 2)


def _sc_gather_body(uidx_h, iidx_h, utab, itab, u_out, i_out,
                    idx_v, rows_v, sem):
  wid = lax.axis_index("s") * NC + lax.axis_index("c")
  base = wid * BPW
  lanes = lax.iota(jnp.int32, 16)
  for idx_h, tab, out in ((uidx_h, utab, u_out), (iidx_h, itab, i_out)):
    pltpu.sync_copy(idx_h.at[wid], idx_v)

    def fire(g, carry):
      vec = idx_v[pl.ds(g * 16, 16)]
      for j in range(16):
        r = jnp.sum(jnp.where(lanes == j, vec, 0))
        pltpu.async_copy(
            tab.at[pl.ds(r, 1)], rows_v.at[pl.ds(g * 16 + j, 1)], sem)
      return carry

    lax.fori_loop(0, BPW // 16, fire, 0)
    pltpu.make_async_copy(tab.at[pl.ds(0, BPW)], rows_v, sem).wait()
    pltpu.sync_copy(rows_v, out.at[pl.ds(base, BPW)])


_sc_gather = functools.partial(
    pl.kernel,
    out_type=(
        jax.ShapeDtypeStruct((B, 2 * D), jnp.uint32),
        jax.ShapeDtypeStruct((B, 2 * D), jnp.uint32),
    ),
    mesh=plsc.VectorSubcoreMesh(core_axis_name="c", subcore_axis_name="s"),
    scratch_types=[
        pltpu.VMEM((BPW,), jnp.int32),
        pltpu.VMEM((BPW, 2 * D), jnp.uint32),
        pltpu.SemaphoreType.DMA,
    ],
    compiler_params=pltpu.CompilerParams(needs_layout_passes=False),
)(_sc_gather_body)


def _unpack_half(x, parity):
  even = lax.bitcast_convert_type(x << 16, jnp.float32)
  odd = lax.bitcast_convert_type(x & jnp.uint32(0xFFFF0000), jnp.float32)
  return jnp.where(parity == 0, even, odd)


def _mlp_body(ub_ref, ib_ref, up_ref, ip_ref, lg_ref, ct_ref,
              lemb_ref, cemb_ref, cwl_ref, cwc_ref, cb_ref,
              w1c_ref, b1_ref, w2t_ref, b2_ref, w3t_ref, b3_ref,
              out_ref):
  h1u = _unpack_half(ub_ref[...], up_ref[...])
  icp = _unpack_half(ib_ref[...], ip_ref[...])
  lw = lemb_ref[...] @ cwl_ref[...]
  cw2 = cemb_ref[...] @ cwc_ref[...]
  ohl = (lg_ref[...] == lax.broadcasted_iota(jnp.int32, (1, NL), 1)
         ).astype(jnp.float32)
  ohc = (ct_ref[...] == lax.broadcasted_iota(jnp.int32, (1, NCAT), 1)
         ).astype(jnp.float32)
  ic = icp[:, :D]
  ic += ohl @ lw
  ic += ohc @ cw2
  ic = jnp.maximum(ic + cb_ref[...], 0.0)
  h1 = h1u
  h1 += ic @ w1c_ref[...]
  h1 = jnp.maximum(h1 + b1_ref[...], 0.0)
  h2 = jnp.maximum(h1 @ w2t_ref[...] + b2_ref[...], 0.0)
  out_ref[...] = h2 @ w3t_ref[...] + b3_ref[...]


def _full(shape):
  return pl.BlockSpec(shape, lambda i: tuple(0 for _ in shape))


_mlp = pl.pallas_call(
    _mlp_body,
    grid=(B // TILE,),
    in_specs=[
        pl.BlockSpec((TILE, 2 * D), lambda i: (i, 0)),
        pl.BlockSpec((TILE, 2 * D), lambda i: (i, 0)),
        pl.BlockSpec((TILE, 1), lambda i: (i, 0)),
        pl.BlockSpec((TILE, 1), lambda i: (i, 0)),
        pl.BlockSpec((TILE, 1), lambda i: (i, 0)),
        pl.BlockSpec((TILE, 1), lambda i: (i, 0)),
        _full((NL, H)),
        _full((NCAT, H)),
        _full((H, D)),
        _full((H, D)),
        _full((1, D)),
        _full((D, 2 * D)),
        _full((1, 2 * D)),
        _full((2 * D, D)),
        _full((1, D)),
        _full((D, 1)),
        _full((1, 1)),
    ],
    out_specs=pl.BlockSpec((TILE, 1), lambda i: (i, 0)),
    out_shape=jax.ShapeDtypeStruct((B, 1), jnp.float32),
    compiler_params=pltpu.CompilerParams(
        dimension_semantics=("arbitrary",)),
)


def kernel(user, item, language, category, user_emb, item_emb, language_emb,
           category_emb, cw, cb, w1, b1, w2, b2, w3, b3):
  w1u = w1[:, :D].T
  cwi_pad = jnp.pad(cw[:, :D].T, ((0, 0), (0, D)))
  u_proj = _proj_u(user_emb.T, w1u)
  i_proj = _proj_i(item_emb.T, cwi_pad)
  u_rows, i_rows = _sc_gather(
      (user // 2).reshape(NW, BPW), (item // 2).reshape(NW, BPW),
      u_proj, i_proj)
  cwl = cw[:, D:D + H].T
  cwc = cw[:, D + H:].T
  w1c = w1[:, D:].T
  out = _mlp(u_rows, i_rows,
             (user % 2).reshape(B, 1), (item % 2).reshape(B, 1),
             language.reshape(B, 1), category.reshape(B, 1),
             language_emb, category_emb,
             cwl, cwc, cb.reshape(1, D),
             w1c, b1.reshape(1, 2 * D),
             w2.T, b2.reshape(1, D),
             w3.T, b3.reshape(1, 1))
  return out[:, 0]


# speedup vs baseline: 1.9116x; 1.9116x over previous
import functools

import jax
import jax.numpy as jnp
from jax import lax
from jax.experimental import pallas as pl
from jax.experimental.pallas import tpu as pltpu
from jax.experimental.pallas import tpu_sc as plsc

B = 16384
NU = 1000000
NI = 100000
NL = 100
NCAT = 1000
D = 64
H = 32

NC = 2
NS = 16
NW = NC * NS
BPW = B // NW

TILE = 512
CKH = 4096


def _proj_body(xt_ref, w_ref, out_ref):
  h = lax.dot_general(
      xt_ref[...], w_ref[...], (((0,), (0,)), ((), ())),
      preferred_element_type=jnp.float32)
  out_ref[...] = pltpu.bitcast(h.astype(jnp.bfloat16), jnp.uint32)


def _make_proj(n_pairs):
  grid = (n_pairs + CKH - 1) // CKH
  return pl.pallas_call(
      _proj_body,
      grid=(grid,),
      in_specs=[
          pl.BlockSpec((D, 2 * CKH), lambda i: (0, i)),
          pl.BlockSpec((D, 2 * D), lambda i: (0, 0)),
      ],
      out_specs=pl.BlockSpec((CKH, 2 * D), lambda i: (i, 0)),
      out_shape=jax.ShapeDtypeStruct((n_pairs, 2 * D), jnp.uint32),
      compiler_params=pltpu.CompilerParams(
          dimension_semantics=("arbitrary",)),
  )


_proj_u = _make_proj(NU // 2)
_proj_i = _make_proj(NI // 2)


def _sc_gather_body(uidx_h, iidx_h, utab, itab, u_out, i_out,
                    idx_v, rows_v, sem):
  wid = lax.axis_index("s") * NC + lax.axis_index("c")
  base = wid * BPW
  lanes = lax.iota(jnp.int32, 16)
  for idx_h, tab, out in ((uidx_h, utab, u_out), (iidx_h, itab, i_out)):
    pltpu.sync_copy(idx_h.at[wid], idx_v)

    def fire(g, carry):
      vec = idx_v[pl.ds(g * 16, 16)]
      for j in range(16):
        r = jnp.sum(jnp.where(lanes == j, vec, 0))
        pltpu.async_copy(
            tab.at[pl.ds(r, 1)], rows_v.at[pl.ds(g * 16 + j, 1)], sem)
      return carry

    lax.fori_loop(0, BPW // 16, fire, 0)
    pltpu.make_async_copy(tab.at[pl.ds(0, BPW)], rows_v, sem).wait()
    pltpu.sync_copy(rows_v, out.at[pl.ds(base, BPW)])


_sc_gather = functools.partial(
    pl.kernel,
    out_type=(
        jax.ShapeDtypeStruct((B, 2 * D), jnp.uint32),
        jax.ShapeDtypeStruct((B, 2 * D), jnp.uint32),
    ),
    mesh=plsc.VectorSubcoreMesh(core_axis_name="c", subcore_axis_name="s"),
    scratch_types=[
        pltpu.VMEM((BPW,), jnp.int32),
        pltpu.VMEM((BPW, 2 * D), jnp.uint32),
        pltpu.SemaphoreType.DMA,
    ],
    compiler_params=pltpu.CompilerParams(needs_layout_passes=False),
)(_sc_gather_body)


def _unpack_half(x, parity):
  even = lax.bitcast_convert_type(x << 16, jnp.float32)
  odd = lax.bitcast_convert_type(x & jnp.uint32(0xFFFF0000), jnp.float32)
  return jnp.where(parity == 0, even, odd)


def _mlp_body(ub_ref, ib_ref, up_ref, ip_ref, lg_ref, ct_ref,
              lemb_ref, cemb_ref, cwl_ref, cwc_ref, cb_ref,
              w1c_ref, b1_ref, w2t_ref, b2_ref, w3t_ref, b3_ref,
              out_ref):
  h1u = _unpack_half(ub_ref[...], up_ref[...])
  icp = _unpack_half(ib_ref[...], ip_ref[...])
  lw = lemb_ref[...] @ cwl_ref[...]
  cw2 = cemb_ref[...] @ cwc_ref[...]
  ohl = (lg_ref[...] == lax.broadcasted_iota(jnp.int32, (1, NL), 1)
         ).astype(jnp.float32)
  ohc = (ct_ref[...] == lax.broadcasted_iota(jnp.int32, (1, NCAT), 1)
         ).astype(jnp.float32)
  ic = icp[:, :D]
  ic += ohl @ lw
  ic += ohc @ cw2
  ic = jnp.maximum(ic + cb_ref[...], 0.0)
  h1 = h1u
  h1 += ic @ w1c_ref[...]
  h1 = jnp.maximum(h1 + b1_ref[...], 0.0)
  h2 = jnp.maximum(h1 @ w2t_ref[...] + b2_ref[...], 0.0)
  out_ref[...] = h2 @ w3t_ref[...] + b3_ref[...]


def _full(shape):
  return pl.BlockSpec(shape, lambda i: tuple(0 for _ in shape))


_mlp = pl.pallas_call(
    _mlp_body,
    grid=(B // TILE,),
    in_specs=[
        pl.BlockSpec((TILE, 2 * D), lambda i: (i, 0)),
        pl.BlockSpec((TILE, 2 * D), lambda i: (i, 0)),
        pl.BlockSpec((TILE, 1), lambda i: (i, 0)),
        pl.BlockSpec((TILE, 1), lambda i: (i, 0)),
        pl.BlockSpec((TILE, 1), lambda i: (i, 0)),
        pl.BlockSpec((TILE, 1), lambda i: (i, 0)),
        _full((NL, H)),
        _full((NCAT, H)),
        _full((H, D)),
        _full((H, D)),
        _full((1, D)),
        _full((D, 2 * D)),
        _full((1, 2 * D)),
        _full((2 * D, D)),
        _full((1, D)),
        _full((D, 1)),
        _full((1, 1)),
    ],
    out_specs=pl.BlockSpec((TILE, 1), lambda i: (i, 0)),
    out_shape=jax.ShapeDtypeStruct((B, 1), jnp.float32),
    compiler_params=pltpu.CompilerParams(
        dimension_semantics=("arbitrary",)),
)


def kernel(user, item, language, category, user_emb, item_emb, language_emb,
           category_emb, cw, cb, w1, b1, w2, b2, w3, b3):
  w1u = w1[:, :D].T
  cwi_pad = jnp.pad(cw[:, :D].T, ((0, 0), (0, D)))
  u_proj = _proj_u(user_emb.T, w1u)
  i_proj = _proj_i(item_emb.T, cwi_pad)
  u_rows, i_rows = _sc_gather(
      (user // 2).reshape(NW, BPW), (item // 2).reshape(NW, BPW),
      u_proj, i_proj)
  cwl = cw[:, D:D + H].T
  cwc = cw[:, D + H:].T
  w1c = w1[:, D:].T
  out = _mlp(u_rows, i_rows,
             (user % 2).reshape(B, 1), (item % 2).reshape(B, 1),
             language.reshape(B, 1), category.reshape(B, 1),
             language_emb, category_emb,
             cwl, cwc, cb.reshape(1, D),
             w1c, b1.reshape(1, 2 * D),
             w2.T, b2.reshape(1, D),
             w3.T, b3.reshape(1, 1))
  return out[:, 0]
